# Initial kernel scaffold; baseline (speedup 1.0000x reference)
import jax
import jax.numpy as jnp
from jax.experimental import pallas as pl


def kernel(xyz, state, msa, pair, seq1hot, idx, CA_atom_index, top_k, params):
    raise NotImplementedError("write your pallas kernel here")


# speedup vs baseline: 2.2365x; 2.2365x over previous
import functools

import jax
import jax.numpy as jnp
from jax import lax
from jax.experimental import pallas as pl
from jax.experimental.pallas import tpu as pltpu
from jax.experimental.pallas import tpu_sc as plsc

L = 1024
K = 64
D_MSA = 256
D_PAIR = 64
D_STATE = 32
D_NODE = 32
D_EDGE = 32
N_RBF = 36

TC1_R = 128
TC2_R = 16
TC2_E = TC2_R * K

_NC, _NS = 2, 16
_NW = _NC * _NS
_CHUNK = 128
_ROWS_PER_W = (L * K) // _NW // _CHUNK
_FIRE = 8


def _ln(x, g, b):
    m = jnp.mean(x, axis=-1, keepdims=True)
    v = jnp.mean((x - m) ** 2, axis=-1, keepdims=True)
    return (x - m) / jnp.sqrt(v + 1e-5) * g + b


def _tc1_body(msa_r, seq_r, st_r, xyzr_r, idxf_r, cat_r,
              lmg_r, lmb_r, lsg_r, lsb_r, wx_r, bx_r, lng_r, lnb_r,
              G_o, nbr_o, nbrd_o, flat_o):
    i = pl.program_id(0)
    msan = _ln(msa_r[...], lmg_r[...], lmb_r[...])
    stn = _ln(st_r[...], lsg_r[...], lsb_r[...])
    nodecat = jnp.concatenate([msan, seq_r[...], stn], axis=1)
    node = jnp.dot(nodecat, wx_r[...], preferred_element_type=jnp.float32) + bx_r[...]
    h = _ln(node, lng_r[...], lnb_r[...])
    xyzb = xyzr_r[...]
    ca_blk = xyzb[:, 3:6]
    v = xyzb - jnp.concatenate([ca_blk, ca_blk, ca_blk], axis=1)
    zeros3 = jnp.zeros((TC1_R, 3), jnp.float32)
    G_o[...] = jnp.concatenate([h, ca_blk, v, idxf_r[...], zeros3], axis=1)

    dx = xyzb[:, 3:4] - cat_r[0:1, :]
    dy = xyzb[:, 4:5] - cat_r[1:2, :]
    dz = xyzb[:, 5:6] - cat_r[2:3, :]
    dist = jnp.sqrt(dx * dx + dy * dy + dz * dz + 1e-8)
    rows = lax.broadcasted_iota(jnp.int32, (TC1_R, L), 0) + i * TC1_R
    cols = lax.broadcasted_iota(jnp.int32, (TC1_R, L), 1)
    dist = jnp.where(rows == cols, dist + 1e9, dist)

    nbrs, dvals = [], []
    for _ in range(K):
        m = jnp.min(dist, axis=1, keepdims=True)
        colc = jnp.where(dist == m, cols, jnp.int32(2 * L))
        c = jnp.min(colc, axis=1, keepdims=True)
        nbrs.append(c)
        dvals.append(m)
        dist = jnp.where(cols == c, jnp.float32(3e38), dist)
    nbr = jnp.concatenate(nbrs, axis=1)
    nbr_o[...] = nbr
    nbrd_o[...] = jnp.concatenate(dvals, axis=1)
    grow = lax.broadcasted_iota(jnp.int32, (TC1_R, K), 0) + i * TC1_R
    flat_o[...] = grow * L + nbr


def _run_tc1(msa, seq1hot, state, xyzr, idxf, ca_t, p):
    full = lambda shp: pl.BlockSpec(shp, lambda i: (0, 0))
    blk = lambda shp: pl.BlockSpec(shp, lambda i: (i, 0))
    return pl.pallas_call(
        _tc1_body,
        grid=(L // TC1_R,),
        in_specs=[
            blk((TC1_R, D_MSA)),
            blk((TC1_R, 21)),
            blk((TC1_R, D_STATE)),
            blk((TC1_R, 9)),
            blk((TC1_R, 1)),
            full((3, L)),
            full((1, D_MSA)), full((1, D_MSA)),
            full((1, D_STATE)), full((1, D_STATE)),
            full((D_MSA + 21 + D_STATE, D_NODE)), full((1, D_NODE)),
            full((1, D_NODE)), full((1, D_NODE)),
        ],
        out_specs=[
            blk((TC1_R, 48)),
            blk((TC1_R, K)),
            blk((TC1_R, K)),
            blk((TC1_R, K)),
        ],
        out_shape=[
            jax.ShapeDtypeStruct((L, 48), jnp.float32),
            jax.ShapeDtypeStruct((L, K), jnp.int32),
            jax.ShapeDtypeStruct((L, K), jnp.float32),
            jax.ShapeDtypeStruct((L, K), jnp.int32),
        ],
    )(msa, seq1hot, state, xyzr, idxf, ca_t,
      p['ln_msa_g'], p['ln_msa_b'], p['ln_state_g'], p['ln_state_b'],
      p['W_x'], p['b_x'], p['ln_node_g'], p['ln_node_b'])


def _sc_gather_body(table_hbm, idx_hbm, out_hbm, idx_v, rows_v, sem):
    wid = lax.axis_index("s") * _NC + lax.axis_index("c")
    base = wid * _ROWS_PER_W
    pltpu.sync_copy(idx_hbm.at[pl.ds(base, _ROWS_PER_W)], idx_v)
    for r in range(_ROWS_PER_W // _FIRE):
        copies = [
            pltpu.async_copy(
                table_hbm.at[idx_v.at[r * _FIRE + j]], rows_v.at[j], sem)
            for j in range(_FIRE)
        ]
        for cp in copies:
            cp.wait()
        pltpu.sync_copy(rows_v, out_hbm.at[pl.ds(base + r * _FIRE, _FIRE)])


@functools.cache
def _make_sc_gather():
    return functools.partial(
        pl.kernel,
        mesh=plsc.VectorSubcoreMesh(
            core_axis_name="c", subcore_axis_name="s", num_cores=_NC),
        out_type=jax.ShapeDtypeStruct((L * K // _CHUNK, _CHUNK, D_PAIR),
                                      jnp.float32),
        scratch_types=[
            pltpu.VMEM((_ROWS_PER_W, _CHUNK), jnp.int32),
            pltpu.VMEM((_FIRE, _CHUNK, D_PAIR), jnp.float32),
            pltpu.SemaphoreType.DMA,
        ],
        compiler_params=pltpu.CompilerParams(use_tc_tiling_on_sc=False),
    )(_sc_gather_body)


def _tc2_body(eraw_r, nbr_r, nbrd_r, own_r, Gf_r, cen_r,
              lpg_r, lpb_r, we1_r, be1_r, l1g_r, l1b_r,
              we2_r, be2_r, l2g_r, l2b_r,
              wm_r, bm_r, pa_r, wh_r, bh_r, wrel_r, wnb_r,
              wo_r, bo_r, lsg_r, lsb_r, wl_r, bl_r,
              xyz_o, lddt_o):
    i = pl.program_id(0)
    nbrc = nbr_r[...]
    onehot = (nbrc == lax.broadcasted_iota(jnp.int32, (TC2_E, L), 1)
              ).astype(jnp.float32)
    gath = jnp.dot(onehot, Gf_r[...], preferred_element_type=jnp.float32)
    h_src = gath[:, 0:32]
    ca_src = gath[:, 32:35]
    v_src = gath[:, 35:44]
    idx_src = gath[:, 44:45]

    own = own_r[...]
    h16 = own[:, 0:32]
    ca16 = own[:, 32:35]
    v16 = own[:, 35:44]
    idx16 = own[:, 44:45]

    def rep(x):
        f = x.shape[1]
        return jnp.broadcast_to(x[:, None, :], (TC2_R, K, f)).reshape(TC2_E, f)

    h_dst = rep(h16)
    ca_dst = rep(ca16)
    idx_dst = rep(idx16)
    relpos = ca_src - ca_dst

    ern = _ln(eraw_r[...], lpg_r[...], lpb_r[...])
    e1 = _ln(jnp.dot(ern, we1_r[...], preferred_element_type=jnp.float32)
             + be1_r[...], l1g_r[...], l1b_r[...])
    d = nbrd_r[...]
    sigma = jnp.float32((22.0 - 2.0) / N_RBF)
    rbf = jnp.exp(-(((d - cen_r[...]) / sigma) ** 2))
    seqsep = jnp.clip(idx_src - idx_dst, -32.0, 32.0) / 32.0
    e2in = jnp.concatenate([e1, rbf, seqsep], axis=1)
    e = _ln(jnp.dot(e2in, we2_r[...], preferred_element_type=jnp.float32)
            + be2_r[...], l2g_r[...], l2b_r[...])

    msgin = jnp.concatenate([h_dst, h_src, e], axis=1)
    mz = jnp.dot(msgin, wm_r[...], preferred_element_type=jnp.float32) + bm_r[...]
    a = pa_r[0, 0]
    msg = jnp.where(mz > 0, mz, a * mz)

    msg_sum = jnp.sum(msg.reshape(TC2_R, K, D_NODE), axis=1)
    cr = jnp.dot(msg, wrel_r[...], preferred_element_type=jnp.float32
                 ).reshape(TC2_R, K, 3)
    cn = jnp.dot(msg, wnb_r[...], preferred_element_type=jnp.float32
                 ).reshape(TC2_R, K, 3)
    rp3 = relpos.reshape(TC2_R, K, 3)
    vs3 = v_src.reshape(TC2_R, K, 9)
    cols9 = []
    for c in range(3):
        for dd in range(3):
            vr = jnp.sum(cr[:, :, c] * rp3[:, :, dd], axis=1, keepdims=True)
            vn = jnp.sum(cn[:, :, c] * vs3[:, :, 3 * c + dd], axis=1,
                         keepdims=True)
            cols9.append(vr + vn)
    acc = jnp.concatenate(cols9, axis=1)
    v_new = v16 + acc / 64.0
    ca_new = ca16 + v_new[:, 3:6]
    grow = lax.broadcasted_iota(jnp.int32, (TC2_R, 1), 0) + i * TC2_R
    ca_set = jnp.where(grow == 0, 0.0, ca_new)
    xyz_o[...] = v_new + jnp.concatenate([ca_set, ca_set, ca_set], axis=1)

    h_new = h16 + jnp.dot(msg_sum, wh_r[...],
                          preferred_element_type=jnp.float32) + bh_r[...]
    shift0 = jnp.dot(h_new, wo_r[...],
                     preferred_element_type=jnp.float32) + bo_r[...]
    z = jnp.dot(_ln(shift0, lsg_r[...], lsb_r[...]), wl_r[...],
                preferred_element_type=jnp.float32) + bl_r[...]
    lddt_o[...] = jax.nn.sigmoid(z)


def _run_tc2(eraw, nbr_col, nbrd_col, G, centers, p):
    full = lambda shp: pl.BlockSpec(shp, lambda i: (0, 0))
    blk = lambda shp: pl.BlockSpec(shp, lambda i: (i, 0))
    return pl.pallas_call(
        _tc2_body,
        grid=(L // TC2_R,),
        in_specs=[
            blk((TC2_E, D_PAIR)),
            blk((TC2_E, 1)),
            blk((TC2_E, 1)),
            blk((TC2_R, 48)),
            full((L, 48)),
            full((1, N_RBF)),
            full((1, D_PAIR)), full((1, D_PAIR)),
            full((D_PAIR, D_EDGE)), full((1, D_EDGE)),
            full((1, D_EDGE)), full((1, D_EDGE)),
            full((D_EDGE + N_RBF + 1, D_EDGE)), full((1, D_EDGE)),
            full((1, D_EDGE)), full((1, D_EDGE)),
            full((2 * D_NODE + D_EDGE, D_NODE)), full((1, D_NODE)),
            full((1, 1)),
            full((D_NODE, D_NODE)), full((1, D_NODE)),
            full((D_NODE, 3)), full((D_NODE, 3)),
            full((D_NODE, D_STATE)), full((1, D_STATE)),
            full((1, D_STATE)), full((1, D_STATE)),
            full((D_STATE, 1)), full((1, 1)),
        ],
        out_specs=[
            blk((TC2_R, 9)),
            blk((TC2_R, 1)),
        ],
        out_shape=[
            jax.ShapeDtypeStruct((L, 9), jnp.float32),
            jax.ShapeDtypeStruct((L, 1), jnp.float32),
        ],
    )(eraw, nbr_col, nbrd_col, G, G, centers,
      p['ln_pair_g'], p['ln_pair_b'], p['W_e1'], p['b_e1'],
      p['ln_e1_g'], p['ln_e1_b'], p['W_e2'], p['b_e2'],
      p['ln_e2_g'], p['ln_e2_b'], p['W_msg'], p['b_msg'], p['prelu_a'],
      p['W_h'], p['b_h'], p['W_rel'], p['W_nb'],
      p['W_out0'], p['b_out0'], p['ln_state_g'], p['ln_state_b'],
      p['W_lddt'], p['b_lddt'])


def _pair_gather(pair_flat, flat_idx):
    idx2d = flat_idx.reshape(L * K // _CHUNK, _CHUNK)
    out = _make_sc_gather()(pair_flat, idx2d)
    return out.reshape(L * K, D_PAIR)


def kernel(xyz, state, msa, pair, seq1hot, idx, CA_atom_index, top_k, params):
    del CA_atom_index, top_k
    p = {k: jnp.asarray(v) for k, v in params.items()}
    for k in list(p):
        if p[k].ndim == 1:
            p[k] = p[k].reshape(1, -1)
        elif p[k].ndim == 0:
            p[k] = p[k].reshape(1, 1)

    xyzr = xyz[0].reshape(L, 9)
    ca_t = xyz[0, :, 1, :].T
    idxf = idx[0].astype(jnp.float32).reshape(L, 1)
    centers = jnp.linspace(2.0, 22.0, N_RBF, dtype=jnp.float32).reshape(1, -1)

    G, nbr, nbrd, flat = _run_tc1(
        msa[0], seq1hot[0], state[0], xyzr, idxf, ca_t, p)

    eraw = _pair_gather(pair.reshape(L * L, D_PAIR), flat.reshape(L * K))

    nbr_col = nbr.reshape(L * K, 1)
    nbrd_col = nbrd.reshape(L * K, 1)
    xyz_o, lddt_o = _run_tc2(eraw, nbr_col, nbrd_col, G, centers, p)

    return xyz_o.reshape(L * 3, 3), lddt_o.reshape(1, L, 1)


# speedup vs baseline: 2.7960x; 1.2502x over previous
import functools

import jax
import jax.numpy as jnp
from jax import lax
from jax.experimental import pallas as pl
from jax.experimental.pallas import tpu as pltpu
from jax.experimental.pallas import tpu_sc as plsc

L = 1024
K = 64
D_MSA = 256
D_PAIR = 64
D_STATE = 32
D_NODE = 32
D_EDGE = 32
N_RBF = 36

TC1_R = 128
TC2_R = 16
TC2_E = TC2_R * K

_NC, _NS = 2, 16
_NW = _NC * _NS
_CHUNK = 128
_ROWS_PER_W = (L * K) // _NW // _CHUNK
_FIRE = 8


def _ln(x, g, b):
    m = jnp.mean(x, axis=-1, keepdims=True)
    v = jnp.mean((x - m) ** 2, axis=-1, keepdims=True)
    return (x - m) / jnp.sqrt(v + 1e-5) * g + b


def _tc1_body(msa_r, seq_r, st_r, xyzr_r, idxf_r, cat_r,
              lmg_r, lmb_r, lsg_r, lsb_r, wx_r, bx_r, lng_r, lnb_r,
              G_o, nbr_o, nbrd_o, flat_o):
    i = pl.program_id(0)
    msan = _ln(msa_r[...], lmg_r[...], lmb_r[...])
    stn = _ln(st_r[...], lsg_r[...], lsb_r[...])
    nodecat = jnp.concatenate([msan, seq_r[...], stn], axis=1)
    node = jnp.dot(nodecat, wx_r[...], preferred_element_type=jnp.float32) + bx_r[...]
    h = _ln(node, lng_r[...], lnb_r[...])
    xyzb = xyzr_r[...]
    ca_blk = xyzb[:, 3:6]
    v = xyzb - jnp.concatenate([ca_blk, ca_blk, ca_blk], axis=1)
    zeros3 = jnp.zeros((TC1_R, 3), jnp.float32)
    G_o[...] = jnp.concatenate([h, ca_blk, v, idxf_r[...], zeros3], axis=1)

    dx = xyzb[:, 3:4] - cat_r[0:1, :]
    dy = xyzb[:, 4:5] - cat_r[1:2, :]
    dz = xyzb[:, 5:6] - cat_r[2:3, :]
    dist = jnp.sqrt(dx * dx + dy * dy + dz * dz + 1e-8)
    rows = lax.broadcasted_iota(jnp.int32, (TC1_R, L), 0) + i * TC1_R
    cols = lax.broadcasted_iota(jnp.int32, (TC1_R, L), 1)
    dist = jnp.where(rows == cols, dist + 1e9, dist)

    nbrs, dvals = [], []
    for _ in range(K):
        m = jnp.min(dist, axis=1, keepdims=True)
        colc = jnp.where(dist == m, cols, jnp.int32(2 * L))
        c = jnp.min(colc, axis=1, keepdims=True)
        nbrs.append(c)
        dvals.append(m)
        dist = jnp.where(cols == c, jnp.float32(3e38), dist)
    nbr = jnp.concatenate(nbrs, axis=1)
    nbr_o[...] = nbr
    nbrd_o[...] = jnp.concatenate(dvals, axis=1)
    grow = lax.broadcasted_iota(jnp.int32, (TC1_R, K), 0) + i * TC1_R
    flat_o[...] = grow * L + nbr


def _run_tc1(msa, seq1hot, state, xyzr, idxf, ca_t, p):
    full = lambda shp: pl.BlockSpec(shp, lambda i: (0, 0))
    blk = lambda shp: pl.BlockSpec(shp, lambda i: (i, 0))
    return pl.pallas_call(
        _tc1_body,
        grid=(L // TC1_R,),
        in_specs=[
            blk((TC1_R, D_MSA)),
            blk((TC1_R, 21)),
            blk((TC1_R, D_STATE)),
            blk((TC1_R, 9)),
            blk((TC1_R, 1)),
            full((3, L)),
            full((1, D_MSA)), full((1, D_MSA)),
            full((1, D_STATE)), full((1, D_STATE)),
            full((D_MSA + 21 + D_STATE, D_NODE)), full((1, D_NODE)),
            full((1, D_NODE)), full((1, D_NODE)),
        ],
        out_specs=[
            blk((TC1_R, 48)),
            blk((TC1_R, K)),
            blk((TC1_R, K)),
            blk((TC1_R, K)),
        ],
        out_shape=[
            jax.ShapeDtypeStruct((L, 48), jnp.float32),
            jax.ShapeDtypeStruct((L, K), jnp.int32),
            jax.ShapeDtypeStruct((L, K), jnp.float32),
            jax.ShapeDtypeStruct((L, K), jnp.int32),
        ],
    )(msa, seq1hot, state, xyzr, idxf, ca_t,
      p['ln_msa_g'], p['ln_msa_b'], p['ln_state_g'], p['ln_state_b'],
      p['W_x'], p['b_x'], p['ln_node_g'], p['ln_node_b'])


def _sc_gather_body(table_hbm, idx_hbm, out_hbm, idx_v, rows_v, sem):
    wid = lax.axis_index("s") * _NC + lax.axis_index("c")
    base = wid * _ROWS_PER_W
    pltpu.sync_copy(idx_hbm.at[pl.ds(base, _ROWS_PER_W)], idx_v)
    for r in range(_ROWS_PER_W // _FIRE):
        copies = [
            pltpu.async_copy(
                table_hbm.at[idx_v.at[r * _FIRE + j]], rows_v.at[j], sem)
            for j in range(_FIRE)
        ]
        for cp in copies:
            cp.wait()
        pltpu.sync_copy(rows_v, out_hbm.at[pl.ds(base + r * _FIRE, _FIRE)])


@functools.cache
def _make_sc_gather():
    return functools.partial(
        pl.kernel,
        mesh=plsc.VectorSubcoreMesh(
            core_axis_name="c", subcore_axis_name="s", num_cores=_NC),
        out_type=jax.ShapeDtypeStruct((L * K // _CHUNK, _CHUNK, D_PAIR),
                                      jnp.float32),
        scratch_types=[
            pltpu.VMEM((_ROWS_PER_W, _CHUNK), jnp.int32),
            pltpu.VMEM((_FIRE, _CHUNK, D_PAIR), jnp.float32),
            pltpu.SemaphoreType.DMA,
        ],
        compiler_params=pltpu.CompilerParams(use_tc_tiling_on_sc=False),
    )(_sc_gather_body)


def _tc2_body(eraw_r, nbr_r, nbrd_r, own_r, Gf_r, cen_r,
              lpg_r, lpb_r, we1_r, be1_r, l1g_r, l1b_r,
              we2_r, be2_r, l2g_r, l2b_r,
              wm_r, bm_r, pa_r, wh_r, bh_r, wrel_r, wnb_r,
              wo_r, bo_r, lsg_r, lsb_r, wl_r, bl_r,
              xyz_o, lddt_o):
    i = pl.program_id(0)
    nbrc = nbr_r[...]
    onehot = (nbrc == lax.broadcasted_iota(jnp.int32, (TC2_E, L), 1)
              ).astype(jnp.float32)
    gath = jnp.dot(onehot, Gf_r[...], preferred_element_type=jnp.float32)
    h_src = gath[:, 0:32]
    ca_src = gath[:, 32:35]
    v_src = gath[:, 35:44]
    idx_src = gath[:, 44:45]

    own = own_r[...]
    h16 = own[:, 0:32]
    ca16 = own[:, 32:35]
    v16 = own[:, 35:44]
    idx16 = own[:, 44:45]

    def rep(x):
        f = x.shape[1]
        return jnp.broadcast_to(x[:, None, :], (TC2_R, K, f)).reshape(TC2_E, f)

    h_dst = rep(h16)
    ca_dst = rep(ca16)
    idx_dst = rep(idx16)
    relpos = ca_src - ca_dst

    ern = _ln(eraw_r[...], lpg_r[...], lpb_r[...])
    e1 = _ln(jnp.dot(ern, we1_r[...], preferred_element_type=jnp.float32)
             + be1_r[...], l1g_r[...], l1b_r[...])
    d = nbrd_r[...]
    sigma = jnp.float32((22.0 - 2.0) / N_RBF)
    rbf = jnp.exp(-(((d - cen_r[...]) / sigma) ** 2))
    seqsep = jnp.clip(idx_src - idx_dst, -32.0, 32.0) / 32.0
    e2in = jnp.concatenate([e1, rbf, seqsep], axis=1)
    e = _ln(jnp.dot(e2in, we2_r[...], preferred_element_type=jnp.float32)
            + be2_r[...], l2g_r[...], l2b_r[...])

    msgin = jnp.concatenate([h_dst, h_src, e], axis=1)
    mz = jnp.dot(msgin, wm_r[...], preferred_element_type=jnp.float32) + bm_r[...]
    a = pa_r[0, 0]
    msg = jnp.where(mz > 0, mz, a * mz)

    msg_sum = jnp.sum(msg.reshape(TC2_R, K, D_NODE), axis=1)
    cr = jnp.dot(msg, wrel_r[...], preferred_element_type=jnp.float32)
    cn = jnp.dot(msg, wnb_r[...], preferred_element_type=jnp.float32)
    rep3 = lambda x: jnp.concatenate(
        [x[:, 0:1], x[:, 0:1], x[:, 0:1],
         x[:, 1:2], x[:, 1:2], x[:, 1:2],
         x[:, 2:3], x[:, 2:3], x[:, 2:3]], axis=1)
    prod = (rep3(cr) * jnp.concatenate([relpos, relpos, relpos], axis=1)
            + rep3(cn) * v_src)
    acc = jnp.sum(prod.reshape(TC2_R, K, 9), axis=1)
    v_new = v16 + acc / 64.0
    ca_new = ca16 + v_new[:, 3:6]
    grow = lax.broadcasted_iota(jnp.int32, (TC2_R, 1), 0) + i * TC2_R
    ca_set = jnp.where(grow == 0, 0.0, ca_new)
    xyz_o[...] = v_new + jnp.concatenate([ca_set, ca_set, ca_set], axis=1)

    h_new = h16 + jnp.dot(msg_sum, wh_r[...],
                          preferred_element_type=jnp.float32) + bh_r[...]
    shift0 = jnp.dot(h_new, wo_r[...],
                     preferred_element_type=jnp.float32) + bo_r[...]
    z = jnp.dot(_ln(shift0, lsg_r[...], lsb_r[...]), wl_r[...],
                preferred_element_type=jnp.float32) + bl_r[...]
    lddt_o[...] = jax.nn.sigmoid(z)


def _run_tc2(eraw, nbr_col, nbrd_col, G, centers, p):
    full = lambda shp: pl.BlockSpec(shp, lambda i: (0, 0))
    blk = lambda shp: pl.BlockSpec(shp, lambda i: (i, 0))
    return pl.pallas_call(
        _tc2_body,
        grid=(L // TC2_R,),
        in_specs=[
            blk((TC2_E, D_PAIR)),
            blk((TC2_E, 1)),
            blk((TC2_E, 1)),
            blk((TC2_R, 48)),
            full((L, 48)),
            full((1, N_RBF)),
            full((1, D_PAIR)), full((1, D_PAIR)),
            full((D_PAIR, D_EDGE)), full((1, D_EDGE)),
            full((1, D_EDGE)), full((1, D_EDGE)),
            full((D_EDGE + N_RBF + 1, D_EDGE)), full((1, D_EDGE)),
            full((1, D_EDGE)), full((1, D_EDGE)),
            full((2 * D_NODE + D_EDGE, D_NODE)), full((1, D_NODE)),
            full((1, 1)),
            full((D_NODE, D_NODE)), full((1, D_NODE)),
            full((D_NODE, 3)), full((D_NODE, 3)),
            full((D_NODE, D_STATE)), full((1, D_STATE)),
            full((1, D_STATE)), full((1, D_STATE)),
            full((D_STATE, 1)), full((1, 1)),
        ],
        out_specs=[
            blk((TC2_R, 9)),
            blk((TC2_R, 1)),
        ],
        out_shape=[
            jax.ShapeDtypeStruct((L, 9), jnp.float32),
            jax.ShapeDtypeStruct((L, 1), jnp.float32),
        ],
    )(eraw, nbr_col, nbrd_col, G, G, centers,
      p['ln_pair_g'], p['ln_pair_b'], p['W_e1'], p['b_e1'],
      p['ln_e1_g'], p['ln_e1_b'], p['W_e2'], p['b_e2'],
      p['ln_e2_g'], p['ln_e2_b'], p['W_msg'], p['b_msg'], p['prelu_a'],
      p['W_h'], p['b_h'], p['W_rel'], p['W_nb'],
      p['W_out0'], p['b_out0'], p['ln_state_g'], p['ln_state_b'],
      p['W_lddt'], p['b_lddt'])


def _pair_gather(pair_flat, flat_idx):
    idx2d = flat_idx.reshape(L * K // _CHUNK, _CHUNK)
    out = _make_sc_gather()(pair_flat, idx2d)
    return out.reshape(L * K, D_PAIR)


def kernel(xyz, state, msa, pair, seq1hot, idx, CA_atom_index, top_k, params):
    del CA_atom_index, top_k
    p = {k: jnp.asarray(v) for k, v in params.items()}
    for k in list(p):
        if p[k].ndim == 1:
            p[k] = p[k].reshape(1, -1)
        elif p[k].ndim == 0:
            p[k] = p[k].reshape(1, 1)

    xyzr = xyz[0].reshape(L, 9)
    ca_t = xyz[0, :, 1, :].T
    idxf = idx[0].astype(jnp.float32).reshape(L, 1)
    centers = jnp.linspace(2.0, 22.0, N_RBF, dtype=jnp.float32).reshape(1, -1)

    G, nbr, nbrd, flat = _run_tc1(
        msa[0], seq1hot[0], state[0], xyzr, idxf, ca_t, p)

    eraw = _pair_gather(pair.reshape(L * L, D_PAIR), flat.reshape(L * K))

    nbr_col = nbr.reshape(L * K, 1)
    nbrd_col = nbrd.reshape(L * K, 1)
    xyz_o, lddt_o = _run_tc2(eraw, nbr_col, nbrd_col, G, centers, p)

    return xyz_o.reshape(L * 3, 3), lddt_o.reshape(1, L, 1)
